# baseline (speedup 1.0000x reference)
import functools

import jax
import jax.numpy as jnp
from jax import lax
from jax.experimental import pallas as pl
from jax.experimental.pallas import tpu as pltpu
from jax.experimental.pallas import tpu_sc as plsc

N = 50000
E = 1600000
D = 64
H = 128
B = 16
MAX_LV = 3

NC = 2
NS = 16
NW = NC * NS
PASSES = D // NW
LANES = 16
CHUNK = 6400
NCH = E // CHUNK
BLK = 512
EBLK = 8192

_mesh = plsc.VectorSubcoreMesh(core_axis_name="c", subcore_axis_name="s")


@functools.partial(
    pl.kernel,
    out_type=jax.ShapeDtypeStruct((D, N), jnp.float32),
    mesh=_mesh,
    compiler_params=pltpu.CompilerParams(needs_layout_passes=False),
    scratch_types=[
        pltpu.VMEM((N,), jnp.float32),
        pltpu.VMEM((N,), jnp.float32),
        pltpu.VMEM((CHUNK,), jnp.int32),
        pltpu.VMEM((CHUNK,), jnp.int32),
        pltpu.SemaphoreType.DMA,
        pltpu.SemaphoreType.DMA,
    ],
)
def _seg_sum_T(h_hbm, pk_hbm, out_hbm, tab, acc, p0, p1, sem0, sem1):
    wid = lax.axis_index("s") * NC + lax.axis_index("c")
    sems = (sem0, sem1)
    pbufs = (p0, p1)

    def issue(k, b):
        pltpu.async_copy(pk_hbm.at[pl.ds(k * CHUNK, CHUNK)], pbufs[b], sems[b])

    def drain(b):
        pltpu.make_async_copy(pk_hbm.at[pl.ds(0, CHUNK)], pbufs[b], sems[b]).wait()

    for p in range(PASSES):
        col = wid + NW * p
        pltpu.sync_copy(h_hbm.at[col], tab)

        @plsc.parallel_loop(0, N, step=LANES, unroll=16)
        def _(i):
            acc[pl.ds(i, LANES)] = jnp.zeros((LANES,), jnp.float32)

        issue(0, 0)
        issue(1, 1)

        def chunk_pair(g, carry):
            for b in range(2):
                k = 2 * g + b
                drain(b)

                def edge_group(j, c):
                    for u in range(16):
                        pk = pbufs[b][pl.ds(j * (16 * LANES) + u * LANES, LANES)]
                        s = pk & 0xFFFF
                        dv = lax.shift_right_logical(pk, 16)
                        vals = plsc.load_gather(tab, [s])
                        plsc.addupdate_scatter(acc, [dv], vals)
                    return c

                lax.fori_loop(0, CHUNK // (16 * LANES), edge_group, 0)

                @pl.when(k + 2 < NCH)
                def _():
                    issue(k + 2, b)

            return carry

        lax.fori_loop(0, NCH // 2, chunk_pair, 0)

        pltpu.sync_copy(acc, out_hbm.at[col])


def _pack_body(ei_ref, out_ref):
    out_ref[...] = (ei_ref[0:1, :] & 0xFFFF) | (ei_ref[1:2, :] << 16)


_pack_edges = pl.pallas_call(
    _pack_body,
    grid=(pl.cdiv(E, EBLK),),
    in_specs=[pl.BlockSpec((2, EBLK), lambda i: (0, i))],
    out_specs=pl.BlockSpec((1, EBLK), lambda i: (0, i)),
    out_shape=jax.ShapeDtypeStruct((1, E), jnp.int32),
)


def _prelude_body(nf_ref, w_ref, msg_ref, h_ref):
    m = jnp.dot(nf_ref[...], w_ref[...], preferred_element_type=jnp.float32)
    msg_ref[...] = m
    h_ref[...] = jnp.maximum(m, 0.0)


_prelude = pl.pallas_call(
    _prelude_body,
    grid=(pl.cdiv(N, BLK),),
    in_specs=[
        pl.BlockSpec((BLK, 2), lambda i: (i, 0)),
        pl.BlockSpec((2, D), lambda i: (0, 0)),
    ],
    out_specs=[
        pl.BlockSpec((BLK, D), lambda i: (i, 0)),
        pl.BlockSpec((BLK, D), lambda i: (i, 0)),
    ],
    out_shape=[
        jax.ShapeDtypeStruct((N, D), jnp.float32),
        jax.ShapeDtypeStruct((N, D), jnp.float32),
    ],
)


def _level_body(pool_ref, w_ref, msg_ref, h_ref):
    m = jnp.dot(pool_ref[...], w_ref[...], preferred_element_type=jnp.float32)
    h_ref[...] = jnp.maximum(m + msg_ref[...], 0.0)


_level = pl.pallas_call(
    _level_body,
    grid=(pl.cdiv(N, BLK),),
    in_specs=[
        pl.BlockSpec((BLK, D), lambda i: (i, 0)),
        pl.BlockSpec((D, D), lambda i: (0, 0)),
        pl.BlockSpec((BLK, D), lambda i: (i, 0)),
    ],
    out_specs=pl.BlockSpec((BLK, D), lambda i: (i, 0)),
    out_shape=jax.ShapeDtypeStruct((N, D), jnp.float32),
)


@functools.partial(
    pl.kernel,
    out_type=jax.ShapeDtypeStruct((D, B), jnp.float32),
    mesh=_mesh,
    compiler_params=pltpu.CompilerParams(needs_layout_passes=False),
    scratch_types=[
        pltpu.VMEM((N,), jnp.float32),
        pltpu.VMEM((N,), jnp.int32),
        pltpu.VMEM((B,), jnp.float32),
    ],
)
def _pool_sc(h_hbm, gid_hbm, out_hbm, tab, gbuf, acc):
    wid = lax.axis_index("s") * NC + lax.axis_index("c")
    pltpu.sync_copy(gid_hbm, gbuf)
    for p in range(PASSES):
        col = wid + NW * p
        pltpu.sync_copy(h_hbm.at[col], tab)
        acc[pl.ds(0, B)] = jnp.zeros((B,), jnp.float32)

        def body(i, c):
            for u in range(5):
                n = i * (5 * LANES) + u * LANES
                g = gbuf[pl.ds(n, LANES)]
                v = tab[pl.ds(n, LANES)]
                plsc.addupdate_scatter(acc, [g], v)
            return c

        lax.fori_loop(0, N // (5 * LANES), body, 0)
        pltpu.sync_copy(acc, out_hbm.at[col])


def _head_body(x_ref, w1_ref, b1_ref, w2_ref, b2_ref, out_ref):
    hid = jnp.dot(x_ref[...], w1_ref[...], preferred_element_type=jnp.float32)
    hid = jnp.maximum(hid + b1_ref[...], 0.0)
    out_ref[...] = jnp.dot(hid, w2_ref[...], preferred_element_type=jnp.float32) + b2_ref[...]


def _make_head(nrows, blk):
    return pl.pallas_call(
        _head_body,
        grid=(pl.cdiv(nrows, blk),),
        in_specs=[
            pl.BlockSpec((blk, D), lambda i: (i, 0)),
            pl.BlockSpec((D, H), lambda i: (0, 0)),
            pl.BlockSpec((1, H), lambda i: (0, 0)),
            pl.BlockSpec((H, 1), lambda i: (0, 0)),
            pl.BlockSpec((1, 1), lambda i: (0, 0)),
        ],
        out_specs=pl.BlockSpec((blk, 1), lambda i: (i, 0)),
        out_shape=jax.ShapeDtypeStruct((nrows, 1), jnp.float32),
    )


_head_nodes = _make_head(N, BLK)
_head_graphs = _make_head(B, B)


def kernel(node_feat, edge_index, graph_ids, W_n2l, W_conv, add_W1, add_b1, add_W2, add_b2):
    ei = edge_index.astype(jnp.int32)
    pk = _pack_edges(ei).reshape(E)
    msg_rows, h_rows = _prelude(node_feat, W_n2l)
    for _ in range(MAX_LV):
        poolT = _seg_sum_T(h_rows.T, pk)
        h_rows = _level(poolT.T, W_conv, msg_rows)
    geT = _pool_sc(h_rows.T, graph_ids.astype(jnp.int32))
    b1r = add_b1.reshape(1, H)
    b2r = add_b2.reshape(1, 1)
    raw_n = _head_nodes(h_rows, add_W1, b1r, add_W2, b2r)
    raw_g = _head_graphs(geT.T, add_W1, b1r, add_W2, b2r)
    return jnp.concatenate([raw_n, raw_g], axis=0)


# speedup vs baseline: 3.0682x; 3.0682x over previous
import functools

import jax
import jax.numpy as jnp
from jax import lax
from jax.experimental import pallas as pl
from jax.experimental.pallas import tpu as pltpu
from jax.experimental.pallas import tpu_sc as plsc

N = 50000
E = 1600000
D = 64
H = 128
B = 16
MAX_LV = 3

NC = 2
NS = 16
NW = NC * NS
PASSES = D // NW
LANES = 16
CHUNK = 10000
NCH = E // CHUNK
BLK = 512
EBLK = 8192

_mesh = plsc.VectorSubcoreMesh(core_axis_name="c", subcore_axis_name="s")


@functools.partial(
    pl.kernel,
    out_type=jax.ShapeDtypeStruct((D, N), jnp.float32),
    mesh=_mesh,
    compiler_params=pltpu.CompilerParams(needs_layout_passes=False),
    scratch_types=[
        pltpu.VMEM((N,), jnp.float32),
        pltpu.VMEM((N,), jnp.float32),
        pltpu.VMEM((CHUNK,), jnp.int32),
        pltpu.VMEM((CHUNK,), jnp.int32),
        pltpu.SemaphoreType.DMA,
        pltpu.SemaphoreType.DMA,
    ],
)
def _seg_sum_T(h_hbm, pk_hbm, out_hbm, tab, acc, p0, p1, sem0, sem1):
    wid = lax.axis_index("s") * NC + lax.axis_index("c")
    sems = (sem0, sem1)
    pbufs = (p0, p1)

    def issue(k, b):
        pltpu.async_copy(pk_hbm.at[pl.ds(k * CHUNK, CHUNK)], pbufs[b], sems[b])

    def drain(b):
        pltpu.make_async_copy(pk_hbm.at[pl.ds(0, CHUNK)], pbufs[b], sems[b]).wait()

    for p in range(PASSES):
        col = wid + NW * p
        pltpu.sync_copy(h_hbm.at[col], tab)

        @plsc.parallel_loop(0, N, step=LANES, unroll=16)
        def _(i):
            acc[pl.ds(i, LANES)] = jnp.zeros((LANES,), jnp.float32)

        issue(0, 0)
        issue(1, 1)

        def chunk_pair(g, carry):
            for b in range(2):
                k = 2 * g + b
                drain(b)

                @plsc.parallel_loop(0, CHUNK, step=LANES, unroll=16)
                def _(j):
                    pk = pbufs[b][pl.ds(j, LANES)]
                    s = pk & 0xFFFF
                    dv = lax.shift_right_logical(pk, 16)
                    vals = plsc.load_gather(tab, [s])
                    plsc.addupdate_scatter(acc, [dv], vals)

                @pl.when(k + 2 < NCH)
                def _():
                    issue(k + 2, b)

            return carry

        lax.fori_loop(0, NCH // 2, chunk_pair, 0)

        pltpu.sync_copy(acc, out_hbm.at[col])


def _pack_body(ei_ref, out_ref):
    out_ref[...] = (ei_ref[0:1, :] & 0xFFFF) | (ei_ref[1:2, :] << 16)


_pack_edges = pl.pallas_call(
    _pack_body,
    grid=(pl.cdiv(E, EBLK),),
    in_specs=[pl.BlockSpec((2, EBLK), lambda i: (0, i))],
    out_specs=pl.BlockSpec((1, EBLK), lambda i: (0, i)),
    out_shape=jax.ShapeDtypeStruct((1, E), jnp.int32),
)


def _prelude_body(nf_ref, w_ref, msg_ref, h_ref):
    m = jnp.dot(nf_ref[...], w_ref[...], preferred_element_type=jnp.float32)
    msg_ref[...] = m
    h_ref[...] = jnp.maximum(m, 0.0)


_prelude = pl.pallas_call(
    _prelude_body,
    grid=(pl.cdiv(N, BLK),),
    in_specs=[
        pl.BlockSpec((BLK, 2), lambda i: (i, 0)),
        pl.BlockSpec((2, D), lambda i: (0, 0)),
    ],
    out_specs=[
        pl.BlockSpec((BLK, D), lambda i: (i, 0)),
        pl.BlockSpec((BLK, D), lambda i: (i, 0)),
    ],
    out_shape=[
        jax.ShapeDtypeStruct((N, D), jnp.float32),
        jax.ShapeDtypeStruct((N, D), jnp.float32),
    ],
)


def _level_body(pool_ref, w_ref, msg_ref, h_ref):
    m = jnp.dot(pool_ref[...], w_ref[...], preferred_element_type=jnp.float32)
    h_ref[...] = jnp.maximum(m + msg_ref[...], 0.0)


_level = pl.pallas_call(
    _level_body,
    grid=(pl.cdiv(N, BLK),),
    in_specs=[
        pl.BlockSpec((BLK, D), lambda i: (i, 0)),
        pl.BlockSpec((D, D), lambda i: (0, 0)),
        pl.BlockSpec((BLK, D), lambda i: (i, 0)),
    ],
    out_specs=pl.BlockSpec((BLK, D), lambda i: (i, 0)),
    out_shape=jax.ShapeDtypeStruct((N, D), jnp.float32),
)


@functools.partial(
    pl.kernel,
    out_type=jax.ShapeDtypeStruct((D, B), jnp.float32),
    mesh=_mesh,
    compiler_params=pltpu.CompilerParams(needs_layout_passes=False),
    scratch_types=[
        pltpu.VMEM((N,), jnp.float32),
        pltpu.VMEM((N,), jnp.int32),
        pltpu.VMEM((B,), jnp.float32),
    ],
)
def _pool_sc(h_hbm, gid_hbm, out_hbm, tab, gbuf, acc):
    wid = lax.axis_index("s") * NC + lax.axis_index("c")
    pltpu.sync_copy(gid_hbm, gbuf)
    for p in range(PASSES):
        col = wid + NW * p
        pltpu.sync_copy(h_hbm.at[col], tab)
        acc[pl.ds(0, B)] = jnp.zeros((B,), jnp.float32)

        def body(i, c):
            for u in range(5):
                n = i * (5 * LANES) + u * LANES
                g = gbuf[pl.ds(n, LANES)]
                v = tab[pl.ds(n, LANES)]
                plsc.addupdate_scatter(acc, [g], v)
            return c

        lax.fori_loop(0, N // (5 * LANES), body, 0)
        pltpu.sync_copy(acc, out_hbm.at[col])


def _head_body(x_ref, w1_ref, b1_ref, w2_ref, b2_ref, out_ref):
    hid = jnp.dot(x_ref[...], w1_ref[...], preferred_element_type=jnp.float32)
    hid = jnp.maximum(hid + b1_ref[...], 0.0)
    out_ref[...] = jnp.dot(hid, w2_ref[...], preferred_element_type=jnp.float32) + b2_ref[...]


def _make_head(nrows, blk):
    return pl.pallas_call(
        _head_body,
        grid=(pl.cdiv(nrows, blk),),
        in_specs=[
            pl.BlockSpec((blk, D), lambda i: (i, 0)),
            pl.BlockSpec((D, H), lambda i: (0, 0)),
            pl.BlockSpec((1, H), lambda i: (0, 0)),
            pl.BlockSpec((H, 1), lambda i: (0, 0)),
            pl.BlockSpec((1, 1), lambda i: (0, 0)),
        ],
        out_specs=pl.BlockSpec((blk, 1), lambda i: (i, 0)),
        out_shape=jax.ShapeDtypeStruct((nrows, 1), jnp.float32),
    )


_head_nodes = _make_head(N, BLK)
_head_graphs = _make_head(B, B)


def kernel(node_feat, edge_index, graph_ids, W_n2l, W_conv, add_W1, add_b1, add_W2, add_b2):
    ei = edge_index.astype(jnp.int32)
    pk = _pack_edges(ei).reshape(E)
    msg_rows, h_rows = _prelude(node_feat, W_n2l)
    for _ in range(MAX_LV):
        poolT = _seg_sum_T(h_rows.T, pk)
        h_rows = _level(poolT.T, W_conv, msg_rows)
    geT = _pool_sc(h_rows.T, graph_ids.astype(jnp.int32))
    b1r = add_b1.reshape(1, H)
    b2r = add_b2.reshape(1, 1)
    raw_n = _head_nodes(h_rows, add_W1, b1r, add_W2, b2r)
    raw_g = _head_graphs(geT.T, add_W1, b1r, add_W2, b2r)
    return jnp.concatenate([raw_n, raw_g], axis=0)


# speedup vs baseline: 3.2497x; 1.0592x over previous
import functools

import jax
import jax.numpy as jnp
from jax import lax
from jax.experimental import pallas as pl
from jax.experimental.pallas import tpu as pltpu
from jax.experimental.pallas import tpu_sc as plsc

N = 50000
E = 1600000
D = 64
H = 128
B = 16
MAX_LV = 3

NC = 2
NS = 16
NW = NC * NS
PASSES = D // NW
LANES = 16
CHUNK = 10000
NCH = E // CHUNK
BLK = 512
EBLK = 8192

_mesh = plsc.VectorSubcoreMesh(core_axis_name="c", subcore_axis_name="s")


@functools.partial(
    pl.kernel,
    out_type=jax.ShapeDtypeStruct((D, N), jnp.float32),
    mesh=_mesh,
    compiler_params=pltpu.CompilerParams(needs_layout_passes=False),
    scratch_types=[
        pltpu.VMEM((N,), jnp.float32),
        pltpu.VMEM((N,), jnp.float32),
        pltpu.VMEM((CHUNK,), jnp.int32),
        pltpu.VMEM((CHUNK,), jnp.int32),
        pltpu.SemaphoreType.DMA,
        pltpu.SemaphoreType.DMA,
    ],
)
def _seg_sum_T(h_hbm, pk_hbm, out_hbm, tab, acc, p0, p1, sem0, sem1):
    wid = lax.axis_index("s") * NC + lax.axis_index("c")
    sems = (sem0, sem1)
    pbufs = (p0, p1)

    def issue(k, b):
        pltpu.async_copy(pk_hbm.at[pl.ds(k * CHUNK, CHUNK)], pbufs[b], sems[b])

    def drain(b):
        pltpu.make_async_copy(pk_hbm.at[pl.ds(0, CHUNK)], pbufs[b], sems[b]).wait()

    for p in range(PASSES):
        col = wid + NW * p
        pltpu.sync_copy(h_hbm.at[col], tab)

        @plsc.parallel_loop(0, N, step=LANES, unroll=16)
        def _(i):
            acc[pl.ds(i, LANES)] = jnp.zeros((LANES,), jnp.float32)

        issue(0, 0)
        issue(1, 1)

        def chunk_pair(g, carry):
            for b in range(2):
                k = 2 * g + b
                drain(b)

                @plsc.parallel_loop(0, CHUNK, step=LANES, unroll=16)
                def _(j):
                    pk = pbufs[b][pl.ds(j, LANES)]
                    s = pk & 0xFFFF
                    dv = lax.shift_right_logical(pk, 16)
                    vals = plsc.load_gather(tab, [s])
                    plsc.addupdate_scatter(acc, [dv], vals)

                @pl.when(k + 2 < NCH)
                def _():
                    issue(k + 2, b)

            return carry

        lax.fori_loop(0, NCH // 2, chunk_pair, 0)

        pltpu.sync_copy(acc, out_hbm.at[col])


def _pack_body(ei_ref, out_ref):
    out_ref[...] = (ei_ref[0:1, :] & 0xFFFF) | (ei_ref[1:2, :] << 16)


_pack_edges = pl.pallas_call(
    _pack_body,
    grid=(pl.cdiv(E, EBLK),),
    in_specs=[pl.BlockSpec((2, EBLK), lambda i: (0, i))],
    out_specs=pl.BlockSpec((1, EBLK), lambda i: (0, i)),
    out_shape=jax.ShapeDtypeStruct((1, E), jnp.int32),
)


def _prelude_body(nf_ref, w_ref, msg_ref, h_ref):
    m = jnp.dot(nf_ref[...], w_ref[...], preferred_element_type=jnp.float32)
    msg_ref[...] = m
    h_ref[...] = jnp.transpose(jnp.maximum(m, 0.0), (1, 0))


_prelude = pl.pallas_call(
    _prelude_body,
    grid=(pl.cdiv(N, BLK),),
    in_specs=[
        pl.BlockSpec((BLK, 2), lambda i: (i, 0)),
        pl.BlockSpec((2, D), lambda i: (0, 0)),
    ],
    out_specs=[
        pl.BlockSpec((BLK, D), lambda i: (i, 0)),
        pl.BlockSpec((D, BLK), lambda i: (0, i)),
    ],
    out_shape=[
        jax.ShapeDtypeStruct((N, D), jnp.float32),
        jax.ShapeDtypeStruct((D, N), jnp.float32),
    ],
)


def _level_body(poolT_ref, w_ref, msg_ref, h_ref):
    pool_rows = jnp.transpose(poolT_ref[...], (1, 0))
    m = jnp.dot(pool_rows, w_ref[...], preferred_element_type=jnp.float32)
    h_ref[...] = jnp.transpose(jnp.maximum(m + msg_ref[...], 0.0), (1, 0))


_level = pl.pallas_call(
    _level_body,
    grid=(pl.cdiv(N, BLK),),
    in_specs=[
        pl.BlockSpec((D, BLK), lambda i: (0, i)),
        pl.BlockSpec((D, D), lambda i: (0, 0)),
        pl.BlockSpec((BLK, D), lambda i: (i, 0)),
    ],
    out_specs=pl.BlockSpec((D, BLK), lambda i: (0, i)),
    out_shape=jax.ShapeDtypeStruct((D, N), jnp.float32),
)


@functools.partial(
    pl.kernel,
    out_type=jax.ShapeDtypeStruct((D, B), jnp.float32),
    mesh=_mesh,
    compiler_params=pltpu.CompilerParams(needs_layout_passes=False),
    scratch_types=[
        pltpu.VMEM((N,), jnp.float32),
        pltpu.VMEM((N,), jnp.int32),
        pltpu.VMEM((B,), jnp.float32),
    ],
)
def _pool_sc(h_hbm, gid_hbm, out_hbm, tab, gbuf, acc):
    wid = lax.axis_index("s") * NC + lax.axis_index("c")
    pltpu.sync_copy(gid_hbm, gbuf)
    for p in range(PASSES):
        col = wid + NW * p
        pltpu.sync_copy(h_hbm.at[col], tab)
        acc[pl.ds(0, B)] = jnp.zeros((B,), jnp.float32)

        def body(i, c):
            for u in range(5):
                n = i * (5 * LANES) + u * LANES
                g = gbuf[pl.ds(n, LANES)]
                v = tab[pl.ds(n, LANES)]
                plsc.addupdate_scatter(acc, [g], v)
            return c

        lax.fori_loop(0, N // (5 * LANES), body, 0)
        pltpu.sync_copy(acc, out_hbm.at[col])


def _head_body(xT_ref, w1_ref, b1_ref, w2_ref, b2_ref, out_ref):
    x = jnp.transpose(xT_ref[...], (1, 0))
    hid = jnp.dot(x, w1_ref[...], preferred_element_type=jnp.float32)
    hid = jnp.maximum(hid + b1_ref[...], 0.0)
    out_ref[...] = jnp.dot(hid, w2_ref[...], preferred_element_type=jnp.float32) + b2_ref[...]


def _make_head(nrows, blk):
    return pl.pallas_call(
        _head_body,
        grid=(pl.cdiv(nrows, blk),),
        in_specs=[
            pl.BlockSpec((D, blk), lambda i: (0, i)),
            pl.BlockSpec((D, H), lambda i: (0, 0)),
            pl.BlockSpec((1, H), lambda i: (0, 0)),
            pl.BlockSpec((H, 1), lambda i: (0, 0)),
            pl.BlockSpec((1, 1), lambda i: (0, 0)),
        ],
        out_specs=pl.BlockSpec((blk, 1), lambda i: (i, 0)),
        out_shape=jax.ShapeDtypeStruct((nrows, 1), jnp.float32),
    )


_head_nodes = _make_head(N, BLK)
_head_graphs = _make_head(B, B)


def kernel(node_feat, edge_index, graph_ids, W_n2l, W_conv, add_W1, add_b1, add_W2, add_b2):
    ei = edge_index.astype(jnp.int32)
    pk = _pack_edges(ei).reshape(E)
    msg_rows, hT = _prelude(node_feat, W_n2l)
    for _ in range(MAX_LV):
        poolT = _seg_sum_T(hT, pk)
        hT = _level(poolT, W_conv, msg_rows)
    geT = _pool_sc(hT, graph_ids.astype(jnp.int32))
    b1r = add_b1.reshape(1, H)
    b2r = add_b2.reshape(1, 1)
    raw_n = _head_nodes(hT, add_W1, b1r, add_W2, b2r)
    raw_g = _head_graphs(geT, add_W1, b1r, add_W2, b2r)
    return jnp.concatenate([raw_n, raw_g], axis=0)
